# Initial kernel scaffold; baseline (speedup 1.0000x reference)
import jax
import jax.numpy as jnp
from jax.experimental import pallas as pl


def kernel(x, edge_index, W1l, b1, W1r, W2l, b2, W2r, W3, b3):
    raise NotImplementedError("write your pallas kernel here")


# speedup vs baseline: 5.7227x; 5.7227x over previous
import functools

import jax
import jax.numpy as jnp
from jax import lax
from jax.experimental import pallas as pl
from jax.experimental.pallas import tpu as pltpu
from jax.experimental.pallas import tpu_sc as plsc

N_NODES = 10000
N_PAD = 10240
N_EDGES = 320000
NC = 2
NS = 16
L = 16
NW = NC * NS
CHUNK = 128
TOTCH = N_EDGES // CHUNK
RPT = N_PAD // NS


def _segsum_body(D, *refs):
    (x_hbm, src_hbm, dst_hbm, agg_hbm, acc_sh, idx_s, idx_d, rows, sem) = refs

    cid = lax.axis_index("c")
    sid = lax.axis_index("s")
    wid = sid * NC + cid

    zvec = jnp.zeros((L,), jnp.float32)

    def zfill(i, _):
        for j in range(D // L):
            rows[i, pl.ds(j * L, L)] = zvec
        return 0

    lax.fori_loop(0, CHUNK, zfill, 0)

    base = sid * RPT
    for r in range(RPT // CHUNK):
        pltpu.sync_copy(rows, acc_sh.at[pl.ds(base + r * CHUNK, CHUNK), :])

    plsc.subcore_barrier()

    nch = (TOTCH - wid + NW - 1) // NW

    def chunk_body(k, _):
        off = pl.multiple_of((wid + k * NW) * CHUNK, CHUNK)
        pltpu.sync_copy(src_hbm.at[pl.ds(off, CHUNK)], idx_s)
        pltpu.sync_copy(dst_hbm.at[pl.ds(off, CHUNK)], idx_d)
        pltpu.async_copy(x_hbm.at[idx_s], rows, sem).wait()
        pltpu.sync_copy(rows, acc_sh.at[idx_d], add=True)
        return 0

    lax.fori_loop(0, nch, chunk_body, 0)

    plsc.subcore_barrier()

    pltpu.sync_copy(acc_sh.at[pl.ds(base, RPT), :],
                    agg_hbm.at[cid, pl.ds(base, RPT), :])


def _make_segsum(D):
    mesh = plsc.VectorSubcoreMesh(core_axis_name="c", subcore_axis_name="s")
    return pl.kernel(
        functools.partial(_segsum_body, D),
        out_type=jax.ShapeDtypeStruct((NC, N_PAD, D), jnp.float32),
        mesh=mesh,
        scratch_types=[
            pltpu.VMEM_SHARED((N_PAD, D), jnp.float32),
            pltpu.VMEM((CHUNK,), jnp.int32),
            pltpu.VMEM((CHUNK,), jnp.int32),
            pltpu.VMEM((CHUNK, D), jnp.float32),
            pltpu.SemaphoreType.DMA,
        ],
        name=f"sc_segsum_d{D}",
    )


_segsum128 = _make_segsum(128)


def _cnt_body(dst_hbm, cnt_hbm, cnt_sh, idx_d, rows, sem):
    cid = lax.axis_index("c")
    sid = lax.axis_index("s")
    wid = sid * NC + cid

    zvec = jnp.zeros((L,), jnp.float32)
    ovec = jnp.ones((L,), jnp.float32)

    def zfill(i, _):
        for j in range(128 // L):
            rows[i, pl.ds(j * L, L)] = zvec
        return 0

    lax.fori_loop(0, CHUNK, zfill, 0)

    base = sid * RPT
    for r in range(RPT // CHUNK):
        pltpu.sync_copy(rows, cnt_sh.at[pl.ds(base + r * CHUNK, CHUNK), :])

    def ofill(i, _):
        for j in range(128 // L):
            rows[i, pl.ds(j * L, L)] = ovec
        return 0

    lax.fori_loop(0, CHUNK, ofill, 0)

    plsc.subcore_barrier()

    nch = (TOTCH - wid + NW - 1) // NW

    def chunk_body(k, _):
        off = pl.multiple_of((wid + k * NW) * CHUNK, CHUNK)
        pltpu.sync_copy(dst_hbm.at[pl.ds(off, CHUNK)], idx_d)
        pltpu.sync_copy(rows, cnt_sh.at[idx_d], add=True)
        return 0

    lax.fori_loop(0, nch, chunk_body, 0)

    plsc.subcore_barrier()

    pltpu.sync_copy(cnt_sh.at[pl.ds(base, RPT), :],
                    cnt_hbm.at[cid, pl.ds(base, RPT), :])


def _make_cnt():
    mesh = plsc.VectorSubcoreMesh(core_axis_name="c", subcore_axis_name="s")
    return pl.kernel(
        _cnt_body,
        out_type=jax.ShapeDtypeStruct((NC, N_PAD, 128), jnp.float32),
        mesh=mesh,
        scratch_types=[
            pltpu.VMEM_SHARED((N_PAD, 128), jnp.float32),
            pltpu.VMEM((CHUNK,), jnp.int32),
            pltpu.VMEM((CHUNK, 128), jnp.float32),
            pltpu.SemaphoreType.DMA,
        ],
        name="sc_degree_cnt",
    )


_cntk = _make_cnt()


_BR = 1000


def _tc1_body(x_r, agg_r, cnt_r, w1l_r, b1_r, w1r_r, h1_r):
    a = agg_r[0] + agg_r[1]
    c = cnt_r[0, :, 0:1] + cnt_r[1, :, 0:1]
    mean = a / jnp.maximum(c, 1.0)
    z = (jnp.dot(mean, w1l_r[...], preferred_element_type=jnp.float32)
         + b1_r[...]
         + jnp.dot(x_r[...], w1r_r[...], preferred_element_type=jnp.float32))
    h1_r[...] = jnp.maximum(z, 0.0)


def _tc2_body(h1_r, agg_r, cnt_r, w2l_r, w2r_r, b2_r, w3_r, b3_r, out_r):
    a = agg_r[0] + agg_r[1]
    c = cnt_r[0, :, 0:1] + cnt_r[1, :, 0:1]
    mean = a / jnp.maximum(c, 1.0)
    z = (jnp.dot(mean, w2l_r[...], preferred_element_type=jnp.float32)
         + b2_r[...]
         + jnp.dot(h1_r[...], w2r_r[...], preferred_element_type=jnp.float32))
    h2 = jnp.maximum(z, 0.0)
    out_r[...] = jnp.dot(h2, w3_r[...], preferred_element_type=jnp.float32) + b3_r[...]


def _full(shape):
    return pl.BlockSpec(shape, lambda i: tuple(0 for _ in shape))


def _tc1(x, aggp, cntp, W1l, b1, W1r):
    grid = N_NODES // _BR
    return pl.pallas_call(
        _tc1_body,
        grid=(grid,),
        in_specs=[
            pl.BlockSpec((_BR, 128), lambda i: (i, 0)),
            pl.BlockSpec((NC, _BR, 128), lambda i: (0, i, 0)),
            pl.BlockSpec((NC, _BR, 128), lambda i: (0, i, 0)),
            _full((128, 128)),
            _full((1, 128)),
            _full((128, 128)),
        ],
        out_specs=pl.BlockSpec((_BR, 128), lambda i: (i, 0)),
        out_shape=jax.ShapeDtypeStruct((N_NODES, 128), jnp.float32),
    )(x, aggp, cntp, W1l, b1.reshape(1, 128), W1r)


def _tc2(h1, aggp, cntp, W2l, W2r, b2, W3, b3):
    grid = N_NODES // _BR
    return pl.pallas_call(
        _tc2_body,
        grid=(grid,),
        in_specs=[
            pl.BlockSpec((_BR, 128), lambda i: (i, 0)),
            pl.BlockSpec((NC, _BR, 128), lambda i: (0, i, 0)),
            pl.BlockSpec((NC, _BR, 128), lambda i: (0, i, 0)),
            _full((128, 64)),
            _full((128, 64)),
            _full((1, 64)),
            _full((64, 1)),
            _full((1, 1)),
        ],
        out_specs=pl.BlockSpec((_BR, 1), lambda i: (i, 0)),
        out_shape=jax.ShapeDtypeStruct((N_NODES, 1), jnp.float32),
    )(h1, aggp, cntp, W2l, W2r, b2.reshape(1, 64), W3, b3.reshape(1, 1))


def kernel(x, edge_index, W1l, b1, W1r, W2l, b2, W2r, W3, b3):
    src = edge_index[0].astype(jnp.int32)
    dst = edge_index[1].astype(jnp.int32)

    agg1p = _segsum128(x, src, dst)[:, :N_NODES]
    cntp = _cntk(dst)[:, :N_NODES]
    h1 = _tc1(x, agg1p, cntp, W1l, b1, W1r)
    agg2p = _segsum128(h1, src, dst)[:, :N_NODES]
    out = _tc2(h1, agg2p, cntp, W2l, W2r, b2, W3, b3)
    return out[:, 0]


# speedup vs baseline: 8.9795x; 1.5691x over previous
import functools

import jax
import jax.numpy as jnp
from jax import lax
from jax.experimental import pallas as pl
from jax.experimental.pallas import tpu as pltpu
from jax.experimental.pallas import tpu_sc as plsc

N_NODES = 10000
N_PAD = 10240
N_EDGES = 320000
NC = 2
NS = 16
L = 16
NW = NC * NS
CHUNK = 128
TOTCH = N_EDGES // CHUNK
RPT = N_PAD // NS


def _segsum_body(D, *refs):
    (x_hbm, src_hbm, dst_hbm, agg_hbm, acc_sh,
     idx_s0, idx_s1, idx_d0, idx_d1, rows0, rows1, sem0, sem1) = refs

    cid = lax.axis_index("c")
    sid = lax.axis_index("s")
    wid = sid * NC + cid

    zvec = jnp.zeros((L,), jnp.float32)

    def zfill(i, _):
        for j in range(D // L):
            rows0[i, pl.ds(j * L, L)] = zvec
        return 0

    lax.fori_loop(0, CHUNK, zfill, 0)

    base = sid * RPT
    for r in range(RPT // CHUNK):
        pltpu.sync_copy(rows0, acc_sh.at[pl.ds(base + r * CHUNK, CHUNK), :])

    plsc.subcore_barrier()

    nch = (TOTCH - wid + NW - 1) // NW

    def _start(c, idx_s, idx_d, rows, sem):
        off = pl.multiple_of((wid + c * NW) * CHUNK, CHUNK)
        pltpu.sync_copy(src_hbm.at[pl.ds(off, CHUNK)], idx_s)
        pltpu.sync_copy(dst_hbm.at[pl.ds(off, CHUNK)], idx_d)
        pltpu.async_copy(x_hbm.at[idx_s], rows, sem)

    def _finish(idx_s, idx_d, rows, sem):
        pltpu.make_async_copy(x_hbm.at[idx_s], rows, sem).wait()
        pltpu.sync_copy(rows, acc_sh.at[idx_d], add=True)

    _start(0, idx_s0, idx_d0, rows0, sem0)

    def pair_body(p, _):
        c1 = 2 * p + 1

        @pl.when(c1 < nch)
        def _():
            _start(c1, idx_s1, idx_d1, rows1, sem1)

        _finish(idx_s0, idx_d0, rows0, sem0)

        @pl.when(c1 + 1 < nch)
        def _():
            _start(c1 + 1, idx_s0, idx_d0, rows0, sem0)

        @pl.when(c1 < nch)
        def _():
            _finish(idx_s1, idx_d1, rows1, sem1)

        return 0

    lax.fori_loop(0, (nch + 1) // 2, pair_body, 0)

    plsc.subcore_barrier()

    pltpu.sync_copy(acc_sh.at[pl.ds(base, RPT), :],
                    agg_hbm.at[cid, pl.ds(base, RPT), :])


def _make_segsum(D):
    mesh = plsc.VectorSubcoreMesh(core_axis_name="c", subcore_axis_name="s")
    return pl.kernel(
        functools.partial(_segsum_body, D),
        out_type=jax.ShapeDtypeStruct((NC, N_PAD, D), jnp.float32),
        mesh=mesh,
        scratch_types=[
            pltpu.VMEM_SHARED((N_PAD, D), jnp.float32),
            pltpu.VMEM((CHUNK,), jnp.int32),
            pltpu.VMEM((CHUNK,), jnp.int32),
            pltpu.VMEM((CHUNK,), jnp.int32),
            pltpu.VMEM((CHUNK,), jnp.int32),
            pltpu.VMEM((CHUNK, D), jnp.float32),
            pltpu.VMEM((CHUNK, D), jnp.float32),
            pltpu.SemaphoreType.DMA,
            pltpu.SemaphoreType.DMA,
        ],
        name=f"sc_segsum_d{D}",
    )


_segsum128 = _make_segsum(128)


def _cnt_body(dst_hbm, cnt_hbm, cnt_sh, idx_d0, idx_d1, rows, sem0, sem1):
    cid = lax.axis_index("c")
    sid = lax.axis_index("s")
    wid = sid * NC + cid

    zvec = jnp.zeros((L,), jnp.float32)
    ovec = jnp.ones((L,), jnp.float32)

    def zfill(i, _):
        for j in range(128 // L):
            rows[i, pl.ds(j * L, L)] = zvec
        return 0

    lax.fori_loop(0, CHUNK, zfill, 0)

    base = sid * RPT
    for r in range(RPT // CHUNK):
        pltpu.sync_copy(rows, cnt_sh.at[pl.ds(base + r * CHUNK, CHUNK), :])

    def ofill(i, _):
        for j in range(128 // L):
            rows[i, pl.ds(j * L, L)] = ovec
        return 0

    lax.fori_loop(0, CHUNK, ofill, 0)

    plsc.subcore_barrier()

    nch = (TOTCH - wid + NW - 1) // NW

    def _start(c, idx_d, sem):
        off = pl.multiple_of((wid + c * NW) * CHUNK, CHUNK)
        pltpu.async_copy(dst_hbm.at[pl.ds(off, CHUNK)], idx_d, sem)

    def _finish(c, idx_d, sem):
        off = pl.multiple_of((wid + c * NW) * CHUNK, CHUNK)
        pltpu.make_async_copy(dst_hbm.at[pl.ds(off, CHUNK)], idx_d, sem).wait()
        pltpu.sync_copy(rows, cnt_sh.at[idx_d], add=True)

    _start(0, idx_d0, sem0)

    def pair_body(p, _):
        c1 = 2 * p + 1

        @pl.when(c1 < nch)
        def _():
            _start(c1, idx_d1, sem1)

        _finish(2 * p, idx_d0, sem0)

        @pl.when(c1 + 1 < nch)
        def _():
            _start(c1 + 1, idx_d0, sem0)

        @pl.when(c1 < nch)
        def _():
            _finish(c1, idx_d1, sem1)

        return 0

    lax.fori_loop(0, (nch + 1) // 2, pair_body, 0)

    plsc.subcore_barrier()

    pltpu.sync_copy(cnt_sh.at[pl.ds(base, RPT), :],
                    cnt_hbm.at[cid, pl.ds(base, RPT), :])


def _make_cnt():
    mesh = plsc.VectorSubcoreMesh(core_axis_name="c", subcore_axis_name="s")
    return pl.kernel(
        _cnt_body,
        out_type=jax.ShapeDtypeStruct((NC, N_PAD, 128), jnp.float32),
        mesh=mesh,
        scratch_types=[
            pltpu.VMEM_SHARED((N_PAD, 128), jnp.float32),
            pltpu.VMEM((CHUNK,), jnp.int32),
            pltpu.VMEM((CHUNK,), jnp.int32),
            pltpu.VMEM((CHUNK, 128), jnp.float32),
            pltpu.SemaphoreType.DMA,
            pltpu.SemaphoreType.DMA,
        ],
        name="sc_degree_cnt",
    )


_cntk = _make_cnt()


_BR = 1000


def _tc1_body(x_r, agg_r, cnt_r, w1l_r, b1_r, w1r_r, h1_r):
    a = agg_r[0] + agg_r[1]
    c = cnt_r[0, :, 0:1] + cnt_r[1, :, 0:1]
    mean = a / jnp.maximum(c, 1.0)
    z = (jnp.dot(mean, w1l_r[...], preferred_element_type=jnp.float32)
         + b1_r[...]
         + jnp.dot(x_r[...], w1r_r[...], preferred_element_type=jnp.float32))
    h1_r[...] = jnp.maximum(z, 0.0)


def _tc2_body(h1_r, agg_r, cnt_r, w2l_r, w2r_r, b2_r, w3_r, b3_r, out_r):
    a = agg_r[0] + agg_r[1]
    c = cnt_r[0, :, 0:1] + cnt_r[1, :, 0:1]
    mean = a / jnp.maximum(c, 1.0)
    z = (jnp.dot(mean, w2l_r[...], preferred_element_type=jnp.float32)
         + b2_r[...]
         + jnp.dot(h1_r[...], w2r_r[...], preferred_element_type=jnp.float32))
    h2 = jnp.maximum(z, 0.0)
    out_r[...] = jnp.dot(h2, w3_r[...], preferred_element_type=jnp.float32) + b3_r[...]


def _full(shape):
    return pl.BlockSpec(shape, lambda i: tuple(0 for _ in shape))


def _tc1(x, aggp, cntp, W1l, b1, W1r):
    grid = N_NODES // _BR
    return pl.pallas_call(
        _tc1_body,
        grid=(grid,),
        in_specs=[
            pl.BlockSpec((_BR, 128), lambda i: (i, 0)),
            pl.BlockSpec((NC, _BR, 128), lambda i: (0, i, 0)),
            pl.BlockSpec((NC, _BR, 128), lambda i: (0, i, 0)),
            _full((128, 128)),
            _full((1, 128)),
            _full((128, 128)),
        ],
        out_specs=pl.BlockSpec((_BR, 128), lambda i: (i, 0)),
        out_shape=jax.ShapeDtypeStruct((N_NODES, 128), jnp.float32),
    )(x, aggp, cntp, W1l, b1.reshape(1, 128), W1r)


def _tc2(h1, aggp, cntp, W2l, W2r, b2, W3, b3):
    grid = N_NODES // _BR
    return pl.pallas_call(
        _tc2_body,
        grid=(grid,),
        in_specs=[
            pl.BlockSpec((_BR, 128), lambda i: (i, 0)),
            pl.BlockSpec((NC, _BR, 128), lambda i: (0, i, 0)),
            pl.BlockSpec((NC, _BR, 128), lambda i: (0, i, 0)),
            _full((128, 64)),
            _full((128, 64)),
            _full((1, 64)),
            _full((64, 1)),
            _full((1, 1)),
        ],
        out_specs=pl.BlockSpec((_BR, 1), lambda i: (i, 0)),
        out_shape=jax.ShapeDtypeStruct((N_NODES, 1), jnp.float32),
    )(h1, aggp, cntp, W2l, W2r, b2.reshape(1, 64), W3, b3.reshape(1, 1))


def kernel(x, edge_index, W1l, b1, W1r, W2l, b2, W2r, W3, b3):
    src = edge_index[0].astype(jnp.int32)
    dst = edge_index[1].astype(jnp.int32)

    agg1p = _segsum128(x, src, dst)
    cntp = _cntk(dst)
    h1 = _tc1(x, agg1p, cntp, W1l, b1, W1r)
    agg2p = _segsum128(h1, src, dst)
    out = _tc2(h1, agg2p, cntp, W2l, W2r, b2, W3, b3)
    return out[:, 0]


# speedup vs baseline: 10.2480x; 1.1413x over previous
import functools

import jax
import jax.numpy as jnp
from jax import lax
from jax.experimental import pallas as pl
from jax.experimental.pallas import tpu as pltpu
from jax.experimental.pallas import tpu_sc as plsc

N_NODES = 10000
N_PAD = 10240
N_EDGES = 320000
NC = 2
NS = 16
L = 16
NW = NC * NS
CHUNK = 128
TOTCH = N_EDGES // CHUNK
RPT = N_PAD // NS


def _segsum_body(D, *refs):
    (x_hbm, src_hbm, dst_hbm, agg_hbm, acc_sh,
     idx_s0, idx_s1, idx_d0, idx_d1, rows0, rows1,
     sem0, sem1, isem0, isem1) = refs

    cid = lax.axis_index("c")
    sid = lax.axis_index("s")
    wid = sid * NC + cid

    zvec = jnp.zeros((L,), jnp.float32)

    def zfill(i, _):
        for j in range(D // L):
            rows0[i, pl.ds(j * L, L)] = zvec
        return 0

    lax.fori_loop(0, CHUNK, zfill, 0)

    base = sid * RPT
    for r in range(RPT // CHUNK):
        pltpu.sync_copy(rows0, acc_sh.at[pl.ds(base + r * CHUNK, CHUNK), :])

    plsc.subcore_barrier()

    nch = (TOTCH - wid + NW - 1) // NW

    def _pf_idx(c, idx_s, idx_d, isem):
        off = pl.multiple_of((wid + c * NW) * CHUNK, CHUNK)
        pltpu.async_copy(src_hbm.at[pl.ds(off, CHUNK)], idx_s, isem)
        pltpu.async_copy(dst_hbm.at[pl.ds(off, CHUNK)], idx_d, isem)

    def _wait_idx(c, idx_s, idx_d, isem):
        off = pl.multiple_of((wid + c * NW) * CHUNK, CHUNK)
        pltpu.make_async_copy(src_hbm.at[pl.ds(off, CHUNK)], idx_s, isem).wait()
        pltpu.make_async_copy(dst_hbm.at[pl.ds(off, CHUNK)], idx_d, isem).wait()

    def _gather(idx_s, rows, gsem):
        pltpu.async_copy(x_hbm.at[idx_s], rows, gsem)

    def _scatter(idx_s, idx_d, rows, gsem):
        pltpu.make_async_copy(x_hbm.at[idx_s], rows, gsem).wait()
        pltpu.sync_copy(rows, acc_sh.at[idx_d], add=True)

    _pf_idx(0, idx_s0, idx_d0, isem0)
    _pf_idx(1, idx_s1, idx_d1, isem1)
    _wait_idx(0, idx_s0, idx_d0, isem0)
    _gather(idx_s0, rows0, sem0)

    def pair_body(p, _):
        c0 = 2 * p
        c1, c2, c3 = c0 + 1, c0 + 2, c0 + 3

        @pl.when(c1 < nch)
        def _():
            _wait_idx(c1, idx_s1, idx_d1, isem1)
            _gather(idx_s1, rows1, sem1)

        _scatter(idx_s0, idx_d0, rows0, sem0)

        @pl.when(c2 < nch)
        def _():
            _pf_idx(c2, idx_s0, idx_d0, isem0)

        @pl.when(c1 < nch)
        def _():
            @pl.when(c2 < nch)
            def _():
                _wait_idx(c2, idx_s0, idx_d0, isem0)
                _gather(idx_s0, rows0, sem0)

            _scatter(idx_s1, idx_d1, rows1, sem1)

            @pl.when(c3 < nch)
            def _():
                _pf_idx(c3, idx_s1, idx_d1, isem1)

        return 0

    lax.fori_loop(0, (nch + 1) // 2, pair_body, 0)

    plsc.subcore_barrier()

    pltpu.sync_copy(acc_sh.at[pl.ds(base, RPT), :],
                    agg_hbm.at[cid, pl.ds(base, RPT), :])


def _make_segsum(D):
    mesh = plsc.VectorSubcoreMesh(core_axis_name="c", subcore_axis_name="s")
    return pl.kernel(
        functools.partial(_segsum_body, D),
        out_type=jax.ShapeDtypeStruct((NC, N_PAD, D), jnp.float32),
        mesh=mesh,
        scratch_types=[
            pltpu.VMEM_SHARED((N_PAD, D), jnp.float32),
            pltpu.VMEM((CHUNK,), jnp.int32),
            pltpu.VMEM((CHUNK,), jnp.int32),
            pltpu.VMEM((CHUNK,), jnp.int32),
            pltpu.VMEM((CHUNK,), jnp.int32),
            pltpu.VMEM((CHUNK, D), jnp.float32),
            pltpu.VMEM((CHUNK, D), jnp.float32),
            pltpu.SemaphoreType.DMA,
            pltpu.SemaphoreType.DMA,
            pltpu.SemaphoreType.DMA,
            pltpu.SemaphoreType.DMA,
        ],
        name=f"sc_segsum_d{D}",
    )


_segsum128 = _make_segsum(128)


def _cnt_body(dst_hbm, cnt_hbm, cnt_sh, idx_d0, idx_d1, rows, sem0, sem1):
    cid = lax.axis_index("c")
    sid = lax.axis_index("s")
    wid = sid * NC + cid

    zvec = jnp.zeros((L,), jnp.float32)
    ovec = jnp.ones((L,), jnp.float32)

    def zfill(i, _):
        for j in range(128 // L):
            rows[i, pl.ds(j * L, L)] = zvec
        return 0

    lax.fori_loop(0, CHUNK, zfill, 0)

    base = sid * RPT
    for r in range(RPT // CHUNK):
        pltpu.sync_copy(rows, cnt_sh.at[pl.ds(base + r * CHUNK, CHUNK), :])

    def ofill(i, _):
        for j in range(128 // L):
            rows[i, pl.ds(j * L, L)] = ovec
        return 0

    lax.fori_loop(0, CHUNK, ofill, 0)

    plsc.subcore_barrier()

    nch = (TOTCH - wid + NW - 1) // NW

    def _start(c, idx_d, sem):
        off = pl.multiple_of((wid + c * NW) * CHUNK, CHUNK)
        pltpu.async_copy(dst_hbm.at[pl.ds(off, CHUNK)], idx_d, sem)

    def _finish(c, idx_d, sem):
        off = pl.multiple_of((wid + c * NW) * CHUNK, CHUNK)
        pltpu.make_async_copy(dst_hbm.at[pl.ds(off, CHUNK)], idx_d, sem).wait()
        pltpu.sync_copy(rows, cnt_sh.at[idx_d], add=True)

    _start(0, idx_d0, sem0)

    def pair_body(p, _):
        c1 = 2 * p + 1

        @pl.when(c1 < nch)
        def _():
            _start(c1, idx_d1, sem1)

        _finish(2 * p, idx_d0, sem0)

        @pl.when(c1 + 1 < nch)
        def _():
            _start(c1 + 1, idx_d0, sem0)

        @pl.when(c1 < nch)
        def _():
            _finish(c1, idx_d1, sem1)

        return 0

    lax.fori_loop(0, (nch + 1) // 2, pair_body, 0)

    plsc.subcore_barrier()

    pltpu.sync_copy(cnt_sh.at[pl.ds(base, RPT), :],
                    cnt_hbm.at[cid, pl.ds(base, RPT), :])


def _make_cnt():
    mesh = plsc.VectorSubcoreMesh(core_axis_name="c", subcore_axis_name="s")
    return pl.kernel(
        _cnt_body,
        out_type=jax.ShapeDtypeStruct((NC, N_PAD, 128), jnp.float32),
        mesh=mesh,
        scratch_types=[
            pltpu.VMEM_SHARED((N_PAD, 128), jnp.float32),
            pltpu.VMEM((CHUNK,), jnp.int32),
            pltpu.VMEM((CHUNK,), jnp.int32),
            pltpu.VMEM((CHUNK, 128), jnp.float32),
            pltpu.SemaphoreType.DMA,
            pltpu.SemaphoreType.DMA,
        ],
        name="sc_degree_cnt",
    )


_cntk = _make_cnt()


_BR = 1000


def _tc1_body(x_r, agg_r, cnt_r, w1l_r, b1_r, w1r_r, h1_r):
    a = agg_r[0] + agg_r[1]
    c = cnt_r[0, :, 0:1] + cnt_r[1, :, 0:1]
    mean = a / jnp.maximum(c, 1.0)
    z = (jnp.dot(mean, w1l_r[...], preferred_element_type=jnp.float32)
         + b1_r[...]
         + jnp.dot(x_r[...], w1r_r[...], preferred_element_type=jnp.float32))
    h1_r[...] = jnp.maximum(z, 0.0)


def _tc2_body(h1_r, agg_r, cnt_r, w2l_r, w2r_r, b2_r, w3_r, b3_r, out_r):
    a = agg_r[0] + agg_r[1]
    c = cnt_r[0, :, 0:1] + cnt_r[1, :, 0:1]
    mean = a / jnp.maximum(c, 1.0)
    z = (jnp.dot(mean, w2l_r[...], preferred_element_type=jnp.float32)
         + b2_r[...]
         + jnp.dot(h1_r[...], w2r_r[...], preferred_element_type=jnp.float32))
    h2 = jnp.maximum(z, 0.0)
    out_r[...] = jnp.dot(h2, w3_r[...], preferred_element_type=jnp.float32) + b3_r[...]


def _full(shape):
    return pl.BlockSpec(shape, lambda i: tuple(0 for _ in shape))


def _tc1(x, aggp, cntp, W1l, b1, W1r):
    grid = N_NODES // _BR
    return pl.pallas_call(
        _tc1_body,
        grid=(grid,),
        in_specs=[
            pl.BlockSpec((_BR, 128), lambda i: (i, 0)),
            pl.BlockSpec((NC, _BR, 128), lambda i: (0, i, 0)),
            pl.BlockSpec((NC, _BR, 128), lambda i: (0, i, 0)),
            _full((128, 128)),
            _full((1, 128)),
            _full((128, 128)),
        ],
        out_specs=pl.BlockSpec((_BR, 128), lambda i: (i, 0)),
        out_shape=jax.ShapeDtypeStruct((N_NODES, 128), jnp.float32),
    )(x, aggp, cntp, W1l, b1.reshape(1, 128), W1r)


def _tc2(h1, aggp, cntp, W2l, W2r, b2, W3, b3):
    grid = N_NODES // _BR
    return pl.pallas_call(
        _tc2_body,
        grid=(grid,),
        in_specs=[
            pl.BlockSpec((_BR, 128), lambda i: (i, 0)),
            pl.BlockSpec((NC, _BR, 128), lambda i: (0, i, 0)),
            pl.BlockSpec((NC, _BR, 128), lambda i: (0, i, 0)),
            _full((128, 64)),
            _full((128, 64)),
            _full((1, 64)),
            _full((64, 1)),
            _full((1, 1)),
        ],
        out_specs=pl.BlockSpec((_BR, 1), lambda i: (i, 0)),
        out_shape=jax.ShapeDtypeStruct((N_NODES, 1), jnp.float32),
    )(h1, aggp, cntp, W2l, W2r, b2.reshape(1, 64), W3, b3.reshape(1, 1))


def kernel(x, edge_index, W1l, b1, W1r, W2l, b2, W2r, W3, b3):
    src = edge_index[0].astype(jnp.int32)
    dst = edge_index[1].astype(jnp.int32)

    agg1p = _segsum128(x, src, dst)
    cntp = _cntk(dst)
    h1 = _tc1(x, agg1p, cntp, W1l, b1, W1r)
    agg2p = _segsum128(h1, src, dst)
    out = _tc2(h1, agg2p, cntp, W2l, W2r, b2, W3, b3)
    return out[:, 0]


# speedup vs baseline: 12.8603x; 1.2549x over previous
import functools

import jax
import jax.numpy as jnp
from jax import lax
from jax.experimental import pallas as pl
from jax.experimental.pallas import tpu as pltpu
from jax.experimental.pallas import tpu_sc as plsc

N_NODES = 10000
N_PAD = 10240
N_EDGES = 320000
NC = 2
NS = 16
L = 16
NW = NC * NS
CHUNK = 128
TOTCH = N_EDGES // CHUNK
RPT = N_PAD // NS


def _segsum_body(D, with_cnt, *refs):
    if with_cnt:
        (x_hbm, src_hbm, dst_hbm, agg_hbm, cnt_hbm, acc_sh, cnt_sh,
         idx_s0, idx_s1, idx_d0, idx_d1, rows0, rows1, ones,
         sem0, sem1, isem0, isem1) = refs
    else:
        (x_hbm, src_hbm, dst_hbm, agg_hbm, acc_sh,
         idx_s0, idx_s1, idx_d0, idx_d1, rows0, rows1,
         sem0, sem1, isem0, isem1) = refs

    cid = lax.axis_index("c")
    sid = lax.axis_index("s")
    wid = sid * NC + cid

    zvec = jnp.zeros((L,), jnp.float32)

    def zfill(i, _):
        for j in range(D // L):
            rows0[i, pl.ds(j * L, L)] = zvec
        if with_cnt:
            ones[i, pl.ds(0, L)] = zvec
        return 0

    lax.fori_loop(0, CHUNK, zfill, 0)

    base = sid * RPT
    for r in range(RPT // CHUNK):
        pltpu.sync_copy(rows0, acc_sh.at[pl.ds(base + r * CHUNK, CHUNK), :])
        if with_cnt:
            pltpu.sync_copy(ones, cnt_sh.at[pl.ds(base + r * CHUNK, CHUNK), :])

    if with_cnt:
        def ofill(i, _):
            ones[i, pl.ds(0, L)] = jnp.ones((L,), jnp.float32)
            return 0

        lax.fori_loop(0, CHUNK, ofill, 0)

    plsc.subcore_barrier()

    nch = (TOTCH - wid + NW - 1) // NW

    def _pf_idx(c, idx_s, idx_d, isem):
        off = pl.multiple_of((wid + c * NW) * CHUNK, CHUNK)
        pltpu.async_copy(src_hbm.at[pl.ds(off, CHUNK)], idx_s, isem)
        pltpu.async_copy(dst_hbm.at[pl.ds(off, CHUNK)], idx_d, isem)

    def _wait_idx(c, idx_s, idx_d, isem):
        off = pl.multiple_of((wid + c * NW) * CHUNK, CHUNK)
        pltpu.make_async_copy(src_hbm.at[pl.ds(off, CHUNK)], idx_s, isem).wait()
        pltpu.make_async_copy(dst_hbm.at[pl.ds(off, CHUNK)], idx_d, isem).wait()

    def _gather(idx_s, rows, gsem):
        pltpu.async_copy(x_hbm.at[idx_s], rows, gsem)

    def _scatter(idx_s, idx_d, rows, gsem):
        pltpu.make_async_copy(x_hbm.at[idx_s], rows, gsem).wait()
        pltpu.sync_copy(rows, acc_sh.at[idx_d], add=True)
        if with_cnt:
            pltpu.sync_copy(ones, cnt_sh.at[idx_d], add=True)

    _pf_idx(0, idx_s0, idx_d0, isem0)
    _pf_idx(1, idx_s1, idx_d1, isem1)
    _wait_idx(0, idx_s0, idx_d0, isem0)
    _gather(idx_s0, rows0, sem0)

    def pair_body(p, _):
        c0 = 2 * p
        c1, c2, c3 = c0 + 1, c0 + 2, c0 + 3

        @pl.when(c1 < nch)
        def _():
            _wait_idx(c1, idx_s1, idx_d1, isem1)
            _gather(idx_s1, rows1, sem1)

        _scatter(idx_s0, idx_d0, rows0, sem0)

        @pl.when(c2 < nch)
        def _():
            _pf_idx(c2, idx_s0, idx_d0, isem0)

        @pl.when(c1 < nch)
        def _():
            @pl.when(c2 < nch)
            def _():
                _wait_idx(c2, idx_s0, idx_d0, isem0)
                _gather(idx_s0, rows0, sem0)

            _scatter(idx_s1, idx_d1, rows1, sem1)

            @pl.when(c3 < nch)
            def _():
                _pf_idx(c3, idx_s1, idx_d1, isem1)

        return 0

    lax.fori_loop(0, (nch + 1) // 2, pair_body, 0)

    plsc.subcore_barrier()

    pltpu.sync_copy(acc_sh.at[pl.ds(base, RPT), :],
                    agg_hbm.at[cid, pl.ds(base, RPT), :])
    if with_cnt:
        pltpu.sync_copy(cnt_sh.at[pl.ds(base, RPT), :],
                        cnt_hbm.at[cid, pl.ds(base, RPT), :])


def _make_segsum(D, with_cnt):
    mesh = plsc.VectorSubcoreMesh(core_axis_name="c", subcore_axis_name="s")
    out_type = [jax.ShapeDtypeStruct((NC, N_PAD, D), jnp.float32)]
    scratch = [pltpu.VMEM_SHARED((N_PAD, D), jnp.float32)]
    if with_cnt:
        out_type.append(jax.ShapeDtypeStruct((NC, N_PAD, L), jnp.float32))
        scratch.append(pltpu.VMEM_SHARED((N_PAD, L), jnp.float32))
    scratch += [
        pltpu.VMEM((CHUNK,), jnp.int32),
        pltpu.VMEM((CHUNK,), jnp.int32),
        pltpu.VMEM((CHUNK,), jnp.int32),
        pltpu.VMEM((CHUNK,), jnp.int32),
        pltpu.VMEM((CHUNK, D), jnp.float32),
        pltpu.VMEM((CHUNK, D), jnp.float32),
    ]
    if with_cnt:
        scratch.append(pltpu.VMEM((CHUNK, L), jnp.float32))
    scratch += [
        pltpu.SemaphoreType.DMA,
        pltpu.SemaphoreType.DMA,
        pltpu.SemaphoreType.DMA,
        pltpu.SemaphoreType.DMA,
    ]
    return pl.kernel(
        functools.partial(_segsum_body, D, with_cnt),
        out_type=out_type,
        mesh=mesh,
        scratch_types=scratch,
        compiler_params=pltpu.CompilerParams(use_tc_tiling_on_sc=False),
        name=f"sc_segsum_d{D}",
    )


_segsum128c = _make_segsum(128, True)
_segsum64 = _make_segsum(64, False)


_BR = 1000


def _tc1_body(x_r, agg_r, cnt_r, w1l_r, b1_r, w1r_r, w2l_r, h1_r, p2_r):
    a = agg_r[0] + agg_r[1]
    c = cnt_r[0, :, 0:1] + cnt_r[1, :, 0:1]
    mean = a / jnp.maximum(c, 1.0)
    z = (jnp.dot(mean, w1l_r[...], preferred_element_type=jnp.float32)
         + b1_r[...]
         + jnp.dot(x_r[...], w1r_r[...], preferred_element_type=jnp.float32))
    h1 = jnp.maximum(z, 0.0)
    h1_r[...] = h1
    p2_r[...] = jnp.dot(h1, w2l_r[...], preferred_element_type=jnp.float32)


def _tc2_body(h1_r, agg_r, cnt_r, w2r_r, b2_r, w3_r, b3_r, out_r):
    a = agg_r[0] + agg_r[1]
    c = cnt_r[0, :, 0:1] + cnt_r[1, :, 0:1]
    mean = a / jnp.maximum(c, 1.0)
    z = (mean
         + b2_r[...]
         + jnp.dot(h1_r[...], w2r_r[...], preferred_element_type=jnp.float32))
    h2 = jnp.maximum(z, 0.0)
    out_r[...] = jnp.dot(h2, w3_r[...], preferred_element_type=jnp.float32) + b3_r[...]


def _full(shape):
    return pl.BlockSpec(shape, lambda i: tuple(0 for _ in shape))


def _tc1(x, aggp, cntp, W1l, b1, W1r, W2l):
    grid = N_NODES // _BR
    return pl.pallas_call(
        _tc1_body,
        grid=(grid,),
        in_specs=[
            pl.BlockSpec((_BR, 128), lambda i: (i, 0)),
            pl.BlockSpec((NC, _BR, 128), lambda i: (0, i, 0)),
            pl.BlockSpec((NC, _BR, L), lambda i: (0, i, 0)),
            _full((128, 128)),
            _full((1, 128)),
            _full((128, 128)),
            _full((128, 64)),
        ],
        out_specs=[
            pl.BlockSpec((_BR, 128), lambda i: (i, 0)),
            pl.BlockSpec((_BR, 64), lambda i: (i, 0)),
        ],
        out_shape=[
            jax.ShapeDtypeStruct((N_NODES, 128), jnp.float32),
            jax.ShapeDtypeStruct((N_NODES, 64), jnp.float32),
        ],
    )(x, aggp, cntp, W1l, b1.reshape(1, 128), W1r, W2l)


def _tc2(h1, aggp, cntp, W2r, b2, W3, b3):
    grid = N_NODES // _BR
    return pl.pallas_call(
        _tc2_body,
        grid=(grid,),
        in_specs=[
            pl.BlockSpec((_BR, 128), lambda i: (i, 0)),
            pl.BlockSpec((NC, _BR, 64), lambda i: (0, i, 0)),
            pl.BlockSpec((NC, _BR, L), lambda i: (0, i, 0)),
            _full((128, 64)),
            _full((1, 64)),
            _full((64, 1)),
            _full((1, 1)),
        ],
        out_specs=pl.BlockSpec((_BR, 1), lambda i: (i, 0)),
        out_shape=jax.ShapeDtypeStruct((N_NODES, 1), jnp.float32),
    )(h1, aggp, cntp, W2r, b2.reshape(1, 64), W3, b3.reshape(1, 1))


def kernel(x, edge_index, W1l, b1, W1r, W2l, b2, W2r, W3, b3):
    src = edge_index[0].astype(jnp.int32)
    dst = edge_index[1].astype(jnp.int32)

    agg1p, cntp = _segsum128c(x, src, dst)
    h1, p2 = _tc1(x, agg1p, cntp, W1l, b1, W1r, W2l)
    (agg2p,) = _segsum64(p2, src, dst)
    out = _tc2(h1, agg2p, cntp, W2r, b2, W3, b3)
    return out[:, 0]


# speedup vs baseline: 12.9111x; 1.0039x over previous
import functools

import jax
import jax.numpy as jnp
from jax import lax
from jax.experimental import pallas as pl
from jax.experimental.pallas import tpu as pltpu
from jax.experimental.pallas import tpu_sc as plsc

N_NODES = 10000
N_PAD = 10240
N_EDGES = 320000
NC = 2
NS = 16
L = 16
NW = NC * NS
CHUNK = 128
TOTCH = N_EDGES // CHUNK
RPT = N_PAD // NS


def _segsum_body(D, with_cnt, *refs):
    if with_cnt:
        (x_hbm, src_hbm, dst_hbm, agg_hbm, cnt_hbm, acc_sh, cnt_sh,
         idx_s0, idx_s1, idx_d0, idx_d1, rows0, rows1, ones,
         sem0, sem1, isem0, isem1) = refs
    else:
        (x_hbm, src_hbm, dst_hbm, agg_hbm, acc_sh,
         idx_s0, idx_s1, idx_d0, idx_d1, rows0, rows1,
         sem0, sem1, isem0, isem1) = refs

    cid = lax.axis_index("c")
    sid = lax.axis_index("s")
    wid = sid * NC + cid

    nch = (TOTCH - wid + NW - 1) // NW

    def _pf_idx(c, idx_s, idx_d, isem):
        off = pl.multiple_of((wid + c * NW) * CHUNK, CHUNK)
        pltpu.async_copy(src_hbm.at[pl.ds(off, CHUNK)], idx_s, isem)
        pltpu.async_copy(dst_hbm.at[pl.ds(off, CHUNK)], idx_d, isem)

    def _wait_idx(c, idx_s, idx_d, isem):
        off = pl.multiple_of((wid + c * NW) * CHUNK, CHUNK)
        pltpu.make_async_copy(src_hbm.at[pl.ds(off, CHUNK)], idx_s, isem).wait()
        pltpu.make_async_copy(dst_hbm.at[pl.ds(off, CHUNK)], idx_d, isem).wait()

    def _gather(idx_s, rows, gsem):
        pltpu.async_copy(x_hbm.at[idx_s], rows, gsem)

    def _scatter(idx_s, idx_d, rows, gsem):
        pltpu.make_async_copy(x_hbm.at[idx_s], rows, gsem).wait()
        pltpu.sync_copy(rows, acc_sh.at[idx_d], add=True)
        if with_cnt:
            pltpu.sync_copy(ones, cnt_sh.at[idx_d], add=True)

    _pf_idx(0, idx_s0, idx_d0, isem0)
    _pf_idx(1, idx_s1, idx_d1, isem1)

    zvec = jnp.zeros((L,), jnp.float32)

    def zfill(i, _):
        for j in range(D // L):
            rows0[i, pl.ds(j * L, L)] = zvec
        if with_cnt:
            ones[i, pl.ds(0, L)] = zvec
        return 0

    lax.fori_loop(0, CHUNK, zfill, 0)

    base = sid * RPT
    for r in range(RPT // CHUNK):
        pltpu.sync_copy(rows0, acc_sh.at[pl.ds(base + r * CHUNK, CHUNK), :])
        if with_cnt:
            pltpu.sync_copy(ones, cnt_sh.at[pl.ds(base + r * CHUNK, CHUNK), :])

    if with_cnt:
        def ofill(i, _):
            ones[i, pl.ds(0, L)] = jnp.ones((L,), jnp.float32)
            return 0

        lax.fori_loop(0, CHUNK, ofill, 0)

    _wait_idx(0, idx_s0, idx_d0, isem0)
    _gather(idx_s0, rows0, sem0)

    plsc.subcore_barrier()

    def pair_body(p, _):
        c0 = 2 * p
        c1, c2, c3 = c0 + 1, c0 + 2, c0 + 3

        @pl.when(c1 < nch)
        def _():
            _wait_idx(c1, idx_s1, idx_d1, isem1)
            _gather(idx_s1, rows1, sem1)

        _scatter(idx_s0, idx_d0, rows0, sem0)

        @pl.when(c2 < nch)
        def _():
            _pf_idx(c2, idx_s0, idx_d0, isem0)

        @pl.when(c1 < nch)
        def _():
            @pl.when(c2 < nch)
            def _():
                _wait_idx(c2, idx_s0, idx_d0, isem0)
                _gather(idx_s0, rows0, sem0)

            _scatter(idx_s1, idx_d1, rows1, sem1)

            @pl.when(c3 < nch)
            def _():
                _pf_idx(c3, idx_s1, idx_d1, isem1)

        return 0

    lax.fori_loop(0, (nch + 1) // 2, pair_body, 0)

    plsc.subcore_barrier()

    pltpu.sync_copy(acc_sh.at[pl.ds(base, RPT), :],
                    agg_hbm.at[cid, pl.ds(base, RPT), :])
    if with_cnt:
        pltpu.sync_copy(cnt_sh.at[pl.ds(base, RPT), :],
                        cnt_hbm.at[cid, pl.ds(base, RPT), :])


def _make_segsum(D, with_cnt):
    mesh = plsc.VectorSubcoreMesh(core_axis_name="c", subcore_axis_name="s")
    out_type = [jax.ShapeDtypeStruct((NC, N_PAD, D), jnp.float32)]
    scratch = [pltpu.VMEM_SHARED((N_PAD, D), jnp.float32)]
    if with_cnt:
        out_type.append(jax.ShapeDtypeStruct((NC, N_PAD, L), jnp.float32))
        scratch.append(pltpu.VMEM_SHARED((N_PAD, L), jnp.float32))
    scratch += [
        pltpu.VMEM((CHUNK,), jnp.int32),
        pltpu.VMEM((CHUNK,), jnp.int32),
        pltpu.VMEM((CHUNK,), jnp.int32),
        pltpu.VMEM((CHUNK,), jnp.int32),
        pltpu.VMEM((CHUNK, D), jnp.float32),
        pltpu.VMEM((CHUNK, D), jnp.float32),
    ]
    if with_cnt:
        scratch.append(pltpu.VMEM((CHUNK, L), jnp.float32))
    scratch += [
        pltpu.SemaphoreType.DMA,
        pltpu.SemaphoreType.DMA,
        pltpu.SemaphoreType.DMA,
        pltpu.SemaphoreType.DMA,
    ]
    return pl.kernel(
        functools.partial(_segsum_body, D, with_cnt),
        out_type=out_type,
        mesh=mesh,
        scratch_types=scratch,
        compiler_params=pltpu.CompilerParams(use_tc_tiling_on_sc=False),
        name=f"sc_segsum_d{D}",
    )


_segsum128c = _make_segsum(128, True)
_segsum64 = _make_segsum(64, False)


_BR = 1000


def _tc1_body(x_r, agg_r, cnt_r, w1l_r, b1_r, w1r_r, w2l_r, h1_r, p2_r):
    a = agg_r[0] + agg_r[1]
    c = cnt_r[0, :, 0:1] + cnt_r[1, :, 0:1]
    mean = a / jnp.maximum(c, 1.0)
    z = (jnp.dot(mean, w1l_r[...], preferred_element_type=jnp.float32)
         + b1_r[...]
         + jnp.dot(x_r[...], w1r_r[...], preferred_element_type=jnp.float32))
    h1 = jnp.maximum(z, 0.0)
    h1_r[...] = h1
    p2_r[...] = jnp.dot(h1, w2l_r[...], preferred_element_type=jnp.float32)


def _tc2_body(h1_r, agg_r, cnt_r, w2r_r, b2_r, w3_r, b3_r, out_r):
    a = agg_r[0] + agg_r[1]
    c = cnt_r[0, :, 0:1] + cnt_r[1, :, 0:1]
    mean = a / jnp.maximum(c, 1.0)
    z = (mean
         + b2_r[...]
         + jnp.dot(h1_r[...], w2r_r[...], preferred_element_type=jnp.float32))
    h2 = jnp.maximum(z, 0.0)
    out_r[...] = jnp.dot(h2, w3_r[...], preferred_element_type=jnp.float32) + b3_r[...]


def _full(shape):
    return pl.BlockSpec(shape, lambda i: tuple(0 for _ in shape))


def _tc1(x, aggp, cntp, W1l, b1, W1r, W2l):
    grid = N_NODES // _BR
    return pl.pallas_call(
        _tc1_body,
        grid=(grid,),
        in_specs=[
            pl.BlockSpec((_BR, 128), lambda i: (i, 0)),
            pl.BlockSpec((NC, _BR, 128), lambda i: (0, i, 0)),
            pl.BlockSpec((NC, _BR, L), lambda i: (0, i, 0)),
            _full((128, 128)),
            _full((1, 128)),
            _full((128, 128)),
            _full((128, 64)),
        ],
        out_specs=[
            pl.BlockSpec((_BR, 128), lambda i: (i, 0)),
            pl.BlockSpec((_BR, 64), lambda i: (i, 0)),
        ],
        out_shape=[
            jax.ShapeDtypeStruct((N_NODES, 128), jnp.float32),
            jax.ShapeDtypeStruct((N_NODES, 64), jnp.float32),
        ],
    )(x, aggp, cntp, W1l, b1.reshape(1, 128), W1r, W2l)


def _tc2(h1, aggp, cntp, W2r, b2, W3, b3):
    grid = N_NODES // _BR
    return pl.pallas_call(
        _tc2_body,
        grid=(grid,),
        in_specs=[
            pl.BlockSpec((_BR, 128), lambda i: (i, 0)),
            pl.BlockSpec((NC, _BR, 64), lambda i: (0, i, 0)),
            pl.BlockSpec((NC, _BR, L), lambda i: (0, i, 0)),
            _full((128, 64)),
            _full((1, 64)),
            _full((64, 1)),
            _full((1, 1)),
        ],
        out_specs=pl.BlockSpec((_BR, 1), lambda i: (i, 0)),
        out_shape=jax.ShapeDtypeStruct((N_NODES, 1), jnp.float32),
    )(h1, aggp, cntp, W2r, b2.reshape(1, 64), W3, b3.reshape(1, 1))


def kernel(x, edge_index, W1l, b1, W1r, W2l, b2, W2r, W3, b3):
    src = edge_index[0].astype(jnp.int32)
    dst = edge_index[1].astype(jnp.int32)

    agg1p, cntp = _segsum128c(x, src, dst)
    h1, p2 = _tc1(x, agg1p, cntp, W1l, b1, W1r, W2l)
    (agg2p,) = _segsum64(p2, src, dst)
    out = _tc2(h1, agg2p, cntp, W2r, b2, W3, b3)
    return out[:, 0]
